# baseline (device time: 35035 ns/iter reference)
import jax
import jax.numpy as jnp
from jax import lax
from jax.experimental import pallas as pl
from jax.experimental.pallas import tpu as pltpu

N_DEV = 4
N_LAYERS = 3
CHUNK = 128


def kernel(x, Win0, Wout0, Win1, Wout1, Win2, Wout2):
    b, d_loc = x.shape
    _, h_dim = Win0.shape
    _, out_loc = Wout0.shape

    def body(x_ref, win0_ref, wout0_ref, win1_ref, wout1_ref, win2_ref,
             wout2_ref, out_ref,
             p4_ref, rs_buf, ag_src, ag_buf,
             rs_sems, ag_sems, rs_send_sems, ag_send_sems, local_sem):
        my = lax.axis_index("i")

        barrier_sem = pltpu.get_barrier_semaphore()
        for j in range(1, N_DEV):
            pl.semaphore_signal(
                barrier_sem, inc=1,
                device_id=((my + j) % N_DEV,),
                device_id_type=pl.DeviceIdType.MESH,
            )
        pl.semaphore_wait(barrier_sem, N_DEV - 1)

        wins = (win0_ref, win1_ref, win2_ref)
        wouts = (wout0_ref, wout1_ref, wout2_ref)

        x_bf = x_ref[:, :].astype(jnp.bfloat16)
        for k in range(N_LAYERS):
            win_bf = wins[k][:, :].astype(jnp.bfloat16)
            wout_bf = wouts[k][:, :].astype(jnp.bfloat16)

            def send_chunk(j):
                def _():
                    pltpu.make_async_remote_copy(
                        src_ref=p4_ref.at[j],
                        dst_ref=rs_buf.at[my],
                        send_sem=rs_send_sems.at[j],
                        recv_sem=rs_sems.at[my],
                        device_id=(j,),
                        device_id_type=pl.DeviceIdType.MESH,
                    ).start()
                return _

            for j in range(N_DEV):
                chunk = jnp.dot(
                    x_bf, win_bf[:, j * CHUNK:(j + 1) * CHUNK],
                    preferred_element_type=jnp.float32,
                ).astype(jnp.bfloat16)
                p4_ref[j] = chunk
                pl.when(j != my)(send_chunk(j))
                @pl.when(j == my)
                def _():
                    pltpu.make_async_copy(
                        p4_ref.at[j], rs_buf.at[j], local_sem
                    ).start()

            for j in range(N_DEV):
                @pl.when(j != my)
                def _():
                    pltpu.make_async_remote_copy(
                        src_ref=p4_ref.at[j],
                        dst_ref=rs_buf.at[j],
                        send_sem=rs_send_sems.at[j],
                        recv_sem=rs_sems.at[j],
                        device_id=(j,),
                        device_id_type=pl.DeviceIdType.MESH,
                    ).wait_recv()
                @pl.when(j == my)
                def _():
                    pltpu.make_async_copy(
                        p4_ref.at[j], rs_buf.at[j], local_sem
                    ).wait()

            acc = (rs_buf[0].astype(jnp.float32)
                   + rs_buf[1].astype(jnp.float32)
                   + rs_buf[2].astype(jnp.float32)
                   + rs_buf[3].astype(jnp.float32))
            ag_src[:, :] = jnp.maximum(acc, 0.0).astype(jnp.bfloat16)

            pltpu.make_async_copy(ag_src, ag_buf.at[my], local_sem).start()
            ag_rdmas = []
            for j in range(1, N_DEV):
                t = (my + j) % N_DEV
                r = pltpu.make_async_remote_copy(
                    src_ref=ag_src,
                    dst_ref=ag_buf.at[my],
                    send_sem=ag_send_sems.at[j - 1],
                    recv_sem=ag_sems.at[my],
                    device_id=(t,),
                    device_id_type=pl.DeviceIdType.MESH,
                )
                r.start()
                ag_rdmas.append(r)

            x_acc = None
            for j in range(N_DEV):
                @pl.when(j != my)
                def _():
                    pltpu.make_async_remote_copy(
                        src_ref=ag_src,
                        dst_ref=ag_buf.at[j],
                        send_sem=ag_send_sems.at[0],
                        recv_sem=ag_sems.at[j],
                        device_id=(j,),
                        device_id_type=pl.DeviceIdType.MESH,
                    ).wait_recv()
                @pl.when(j == my)
                def _():
                    pltpu.make_async_copy(
                        ag_src, ag_buf.at[j], local_sem
                    ).wait()
                term = jnp.dot(
                    ag_buf[j], wout_bf[j * CHUNK:(j + 1) * CHUNK, :],
                    preferred_element_type=jnp.float32,
                )
                x_acc = term if x_acc is None else x_acc + term

            for j in range(N_DEV):
                @pl.when(j != my)
                def _():
                    pltpu.make_async_remote_copy(
                        src_ref=p4_ref.at[j],
                        dst_ref=rs_buf.at[my],
                        send_sem=rs_send_sems.at[j],
                        recv_sem=rs_sems.at[my],
                        device_id=(j,),
                        device_id_type=pl.DeviceIdType.MESH,
                    ).wait_send()
            for r in ag_rdmas:
                r.wait_send()

            if k + 1 < N_LAYERS:
                x_bf = x_acc.astype(jnp.bfloat16)
        out_ref[:, :] = x_acc

    return pl.pallas_call(
        body,
        out_shape=jax.ShapeDtypeStruct((b, out_loc), jnp.float32),
        in_specs=[pl.BlockSpec(memory_space=pltpu.VMEM)] * 7,
        out_specs=pl.BlockSpec(memory_space=pltpu.VMEM),
        scratch_shapes=[
            pltpu.VMEM((N_DEV, b, CHUNK), jnp.bfloat16),
            pltpu.VMEM((N_DEV, b, CHUNK), jnp.bfloat16),
            pltpu.VMEM((b, CHUNK), jnp.bfloat16),
            pltpu.VMEM((N_DEV, b, CHUNK), jnp.bfloat16),
            pltpu.SemaphoreType.DMA((N_DEV,)),
            pltpu.SemaphoreType.DMA((N_DEV,)),
            pltpu.SemaphoreType.DMA((N_DEV,)),
            pltpu.SemaphoreType.DMA((N_DEV - 1,)),
            pltpu.SemaphoreType.DMA,
        ],
        compiler_params=pltpu.CompilerParams(collective_id=0),
    )(x, Win0, Wout0, Win1, Wout1, Win2, Wout2)


# device time: 30800 ns/iter; 1.1375x vs baseline; 1.1375x over previous
import jax
import jax.numpy as jnp
from jax import lax
from jax.experimental import pallas as pl
from jax.experimental.pallas import tpu as pltpu

N_DEV = 4
N_LAYERS = 3
HALF = 128


def kernel(x, Win0, Wout0, Win1, Wout1, Win2, Wout2):
    b, d_loc = x.shape
    _, h_dim = Win0.shape
    _, out_loc = Wout0.shape

    def body(x_ref, win0_ref, wout0_ref, win1_ref, wout1_ref, win2_ref,
             wout2_ref, out_ref,
             psrc, pbuf, recv_sems, send_sems, lsems):
        my = lax.axis_index("i")

        barrier_sem = pltpu.get_barrier_semaphore()
        for j in range(1, N_DEV):
            pl.semaphore_signal(
                barrier_sem, inc=1,
                device_id=((my + j) % N_DEV,),
                device_id_type=pl.DeviceIdType.MESH,
            )
        pl.semaphore_wait(barrier_sem, N_DEV - 1)

        wins = (win0_ref, win1_ref, win2_ref)
        wouts = (wout0_ref, wout1_ref, wout2_ref)

        win_bf_cache = {}
        wout_bf_cache = {}

        def win_bf(k):
            if k not in win_bf_cache:
                win_bf_cache[k] = wins[k][:, :].astype(jnp.bfloat16)
            return win_bf_cache[k]

        def wout_bf(k):
            if k not in wout_bf_cache:
                wout_bf_cache[k] = wouts[k][:, :].astype(jnp.bfloat16)
            return wout_bf_cache[k]

        def send_descs(k, h):
            p = k % 2
            descs = []
            for j in range(1, N_DEV):
                t = (my + j) % N_DEV
                descs.append(pltpu.make_async_remote_copy(
                    src_ref=psrc.at[h],
                    dst_ref=pbuf.at[p, h, my],
                    send_sem=send_sems.at[h, j - 1],
                    recv_sem=recv_sems.at[p, h, my],
                    device_id=(t,),
                    device_id_type=pl.DeviceIdType.MESH,
                ))
            return descs

        xh = [x_ref[0:HALF, :].astype(jnp.bfloat16),
              x_ref[HALF:2 * HALF, :].astype(jnp.bfloat16)]

        def issue(k, h):
            p = k % 2
            if k > 0:
                for d in send_descs(k - 1, h):
                    d.wait_send()
            partial = jnp.dot(
                xh[h], win_bf(k), preferred_element_type=jnp.float32
            )
            psrc[h] = partial.astype(jnp.bfloat16)
            pltpu.make_async_copy(
                psrc.at[h], pbuf.at[p, h, my], lsems.at[h]
            ).start()
            for d in send_descs(k, h):
                d.start()

        def consume(k, h):
            p = k % 2
            for j in range(1, N_DEV):
                s = (my + j) % N_DEV
                pltpu.make_async_remote_copy(
                    src_ref=psrc.at[h],
                    dst_ref=pbuf.at[p, h, s],
                    send_sem=send_sems.at[h, 0],
                    recv_sem=recv_sems.at[p, h, s],
                    device_id=(s,),
                    device_id_type=pl.DeviceIdType.MESH,
                ).wait_recv()
            pltpu.make_async_copy(
                psrc.at[h], pbuf.at[p, h, my], lsems.at[h]
            ).wait()
            acc = (pbuf[p, h, 0].astype(jnp.float32)
                   + pbuf[p, h, 1].astype(jnp.float32)
                   + pbuf[p, h, 2].astype(jnp.float32)
                   + pbuf[p, h, 3].astype(jnp.float32))
            relu_bf = jnp.maximum(acc, 0.0).astype(jnp.bfloat16)
            x_new = jnp.dot(
                relu_bf, wout_bf(k), preferred_element_type=jnp.float32
            )
            if k + 1 < N_LAYERS:
                xh[h] = x_new.astype(jnp.bfloat16)
            else:
                out_ref[h * HALF:(h + 1) * HALF, :] = x_new

        issue(0, 0)
        issue(0, 1)
        for k in range(N_LAYERS):
            for h in (0, 1):
                consume(k, h)
                if k + 1 < N_LAYERS:
                    issue(k + 1, h)
        for h in (0, 1):
            for d in send_descs(N_LAYERS - 1, h):
                d.wait_send()

    return pl.pallas_call(
        body,
        out_shape=jax.ShapeDtypeStruct((b, out_loc), jnp.float32),
        in_specs=[pl.BlockSpec(memory_space=pltpu.VMEM)] * 7,
        out_specs=pl.BlockSpec(memory_space=pltpu.VMEM),
        scratch_shapes=[
            pltpu.VMEM((2, HALF, h_dim), jnp.bfloat16),
            pltpu.VMEM((2, 2, N_DEV, HALF, h_dim), jnp.bfloat16),
            pltpu.SemaphoreType.DMA((2, 2, N_DEV)),
            pltpu.SemaphoreType.DMA((2, N_DEV - 1)),
            pltpu.SemaphoreType.DMA((2,)),
        ],
        compiler_params=pltpu.CompilerParams(collective_id=0),
    )(x, Win0, Wout0, Win1, Wout1, Win2, Wout2)


# device time: 30541 ns/iter; 1.1471x vs baseline; 1.0085x over previous
import jax
import jax.numpy as jnp
from jax import lax
from jax.experimental import pallas as pl
from jax.experimental.pallas import tpu as pltpu

N_DEV = 4
N_LAYERS = 3
HALF = 128


def kernel(x, Win0, Wout0, Win1, Wout1, Win2, Wout2):
    b, d_loc = x.shape
    _, h_dim = Win0.shape
    _, out_loc = Wout0.shape

    def body(x_ref, win0_ref, wout0_ref, win1_ref, wout1_ref, win2_ref,
             wout2_ref, out_ref,
             psrc, pbuf, recv_sems, send_sems):
        my = lax.axis_index("i")

        barrier_sem = pltpu.get_barrier_semaphore()
        for j in range(1, N_DEV):
            pl.semaphore_signal(
                barrier_sem, inc=1,
                device_id=((my + j) % N_DEV,),
                device_id_type=pl.DeviceIdType.MESH,
            )
        pl.semaphore_wait(barrier_sem, N_DEV - 1)

        wins = (win0_ref, win1_ref, win2_ref)
        wouts = (wout0_ref, wout1_ref, wout2_ref)

        def send_descs(k, h):
            p = k % 2
            descs = []
            for j in range(1, N_DEV):
                t = (my + j) % N_DEV
                descs.append(pltpu.make_async_remote_copy(
                    src_ref=psrc.at[h],
                    dst_ref=pbuf.at[p, h, 3 - j],
                    send_sem=send_sems.at[h, j - 1],
                    recv_sem=recv_sems.at[p, h, 3 - j],
                    device_id=(t,),
                    device_id_type=pl.DeviceIdType.MESH,
                ))
            return descs

        xh = [x_ref[0:HALF, :].astype(jnp.bfloat16),
              x_ref[HALF:2 * HALF, :].astype(jnp.bfloat16)]
        own_partial = [None, None]
        win_bf = {}
        wout_bf = {}

        def get_win(k):
            if k not in win_bf:
                win_bf[k] = wins[k][:, :].astype(jnp.bfloat16)
            return win_bf[k]

        def get_wout(k):
            if k not in wout_bf:
                wout_bf[k] = wouts[k][:, :].astype(jnp.bfloat16)
            return wout_bf[k]

        def issue(k, h):
            if k > 0:
                for d in send_descs(k - 1, h):
                    d.wait_send()
            partial = jnp.dot(
                xh[h], get_win(k), preferred_element_type=jnp.float32
            )
            own_partial[h] = partial
            psrc[h] = partial.astype(jnp.bfloat16)
            for d in send_descs(k, h):
                d.start()

        def consume(k, h):
            p = k % 2
            for r in range(3):
                pltpu.make_async_remote_copy(
                    src_ref=psrc.at[h],
                    dst_ref=pbuf.at[p, h, r],
                    send_sem=send_sems.at[h, 0],
                    recv_sem=recv_sems.at[p, h, r],
                    device_id=((my + 1) % N_DEV,),
                    device_id_type=pl.DeviceIdType.MESH,
                ).wait_recv()
            acc = (own_partial[h]
                   + pbuf[p, h, 0].astype(jnp.float32)
                   + pbuf[p, h, 1].astype(jnp.float32)
                   + pbuf[p, h, 2].astype(jnp.float32))
            relu_bf = jnp.maximum(acc, 0.0).astype(jnp.bfloat16)
            x_new = jnp.dot(
                relu_bf, get_wout(k), preferred_element_type=jnp.float32
            )
            if k + 1 < N_LAYERS:
                xh[h] = x_new.astype(jnp.bfloat16)
            else:
                out_ref[h * HALF:(h + 1) * HALF, :] = x_new

        issue(0, 0)
        issue(0, 1)
        for k in range(N_LAYERS):
            get_win(k)
            get_wout(k)
        for k in range(N_LAYERS):
            for h in (0, 1):
                consume(k, h)
                if k + 1 < N_LAYERS:
                    issue(k + 1, h)
        for h in (0, 1):
            for d in send_descs(N_LAYERS - 1, h):
                d.wait_send()

    return pl.pallas_call(
        body,
        out_shape=jax.ShapeDtypeStruct((b, out_loc), jnp.float32),
        in_specs=[pl.BlockSpec(memory_space=pltpu.VMEM)] * 7,
        out_specs=pl.BlockSpec(memory_space=pltpu.VMEM),
        scratch_shapes=[
            pltpu.VMEM((2, HALF, h_dim), jnp.bfloat16),
            pltpu.VMEM((2, 2, 3, HALF, h_dim), jnp.bfloat16),
            pltpu.SemaphoreType.DMA((2, 2, 3)),
            pltpu.SemaphoreType.DMA((2, N_DEV - 1)),
        ],
        compiler_params=pltpu.CompilerParams(collective_id=0),
    )(x, Win0, Wout0, Win1, Wout1, Win2, Wout2)


# device time: 29392 ns/iter; 1.1920x vs baseline; 1.0391x over previous
import jax
import jax.numpy as jnp
from jax import lax
from jax.experimental import pallas as pl
from jax.experimental.pallas import tpu as pltpu

N_DEV = 4
N_LAYERS = 3
HALF = 128


def kernel(x, Win0, Wout0, Win1, Wout1, Win2, Wout2):
    b, d_loc = x.shape
    _, h_dim = Win0.shape
    _, out_loc = Wout0.shape

    def body(x_ref, win0_ref, wout0_ref, win1_ref, wout1_ref, win2_ref,
             wout2_ref, out_ref,
             psrc, pbuf, recv_sems, send_sems):
        my = lax.axis_index("i")

        barrier_sem = pltpu.get_barrier_semaphore()
        for j in range(1, N_DEV):
            pl.semaphore_signal(
                barrier_sem, inc=1,
                device_id=((my + j) % N_DEV,),
                device_id_type=pl.DeviceIdType.MESH,
            )
        pl.semaphore_wait(barrier_sem, N_DEV - 1)

        wins = (win0_ref, win1_ref, win2_ref)
        wouts = (wout0_ref, wout1_ref, wout2_ref)

        def send_descs(k, h):
            p = k % 2
            descs = []
            for j in (2, 1, 3):
                t = (my + j) % N_DEV
                descs.append(pltpu.make_async_remote_copy(
                    src_ref=psrc.at[h],
                    dst_ref=pbuf.at[p, h, 3 - j],
                    send_sem=send_sems.at[h, j - 1],
                    recv_sem=recv_sems.at[p, h, 3 - j],
                    device_id=(t,),
                    device_id_type=pl.DeviceIdType.MESH,
                ))
            return descs

        xh = [x_ref[0:HALF, :].astype(jnp.bfloat16),
              x_ref[HALF:2 * HALF, :].astype(jnp.bfloat16)]
        own_partial = [None, None]
        win_bf = {}
        wout_bf = {}

        def get_win(k):
            if k not in win_bf:
                win_bf[k] = wins[k][:, :].astype(jnp.bfloat16)
            return win_bf[k]

        def get_wout(k):
            if k not in wout_bf:
                wout_bf[k] = wouts[k][:, :].astype(jnp.bfloat16)
            return wout_bf[k]

        def issue(k, h):
            if k > 0:
                for d in send_descs(k - 1, h):
                    d.wait_send()
            partial = jnp.dot(
                xh[h], get_win(k), preferred_element_type=jnp.float32
            )
            own_partial[h] = partial
            psrc[h] = partial.astype(jnp.bfloat16)
            for d in send_descs(k, h):
                d.start()

        def consume(k, h):
            p = k % 2
            for r in range(3):
                pltpu.make_async_remote_copy(
                    src_ref=psrc.at[h],
                    dst_ref=pbuf.at[p, h, r],
                    send_sem=send_sems.at[h, 0],
                    recv_sem=recv_sems.at[p, h, r],
                    device_id=((my + 1) % N_DEV,),
                    device_id_type=pl.DeviceIdType.MESH,
                ).wait_recv()
            acc = (own_partial[h]
                   + pbuf[p, h, 0].astype(jnp.float32)
                   + pbuf[p, h, 1].astype(jnp.float32)
                   + pbuf[p, h, 2].astype(jnp.float32))
            relu_bf = jnp.maximum(acc, 0.0).astype(jnp.bfloat16)
            x_new = jnp.dot(
                relu_bf, get_wout(k), preferred_element_type=jnp.float32
            )
            if k + 1 < N_LAYERS:
                xh[h] = x_new.astype(jnp.bfloat16)
            else:
                out_ref[h * HALF:(h + 1) * HALF, :] = x_new

        issue(0, 0)
        issue(0, 1)
        for k in range(N_LAYERS):
            get_win(k)
            get_wout(k)
        for k in range(N_LAYERS):
            for h in (0, 1):
                consume(k, h)
                if k + 1 < N_LAYERS:
                    issue(k + 1, h)
        for h in (0, 1):
            for d in send_descs(N_LAYERS - 1, h):
                d.wait_send()

    return pl.pallas_call(
        body,
        out_shape=jax.ShapeDtypeStruct((b, out_loc), jnp.float32),
        in_specs=[pl.BlockSpec(memory_space=pltpu.VMEM)] * 7,
        out_specs=pl.BlockSpec(memory_space=pltpu.VMEM),
        scratch_shapes=[
            pltpu.VMEM((2, HALF, h_dim), jnp.bfloat16),
            pltpu.VMEM((2, 2, 3, HALF, h_dim), jnp.bfloat16),
            pltpu.SemaphoreType.DMA((2, 2, 3)),
            pltpu.SemaphoreType.DMA((2, N_DEV - 1)),
        ],
        compiler_params=pltpu.CompilerParams(collective_id=0),
    )(x, Win0, Wout0, Win1, Wout1, Win2, Wout2)
